# Initial kernel scaffold; baseline (speedup 1.0000x reference)
import jax
import jax.numpy as jnp
from jax.experimental import pallas as pl


def kernel(x, edge_index, edge_type, basis, att, self_weight, bias):
    raise NotImplementedError("write your pallas kernel here")


# speedup vs baseline: 33.2729x; 33.2729x over previous
import functools

import jax
import jax.numpy as jnp
from jax import lax
from jax.experimental import pallas as pl
from jax.experimental.pallas import tpu as pltpu
from jax.experimental.pallas import tpu_sc as plsc

_PREC = lax.Precision.HIGHEST


def _w_body(att_ref, basis_ref, w_ref):
    w_ref[...] = lax.dot_general(
        att_ref[...], basis_ref[...], (((1,), (0,)), ((), ())),
        preferred_element_type=jnp.float32, precision=_PREC)


def _h_body(x_ref, w_ref, h_ref):
    h_ref[...] = jnp.dot(x_ref[...], w_ref[0],
                         preferred_element_type=jnp.float32,
                         precision=_PREC)[None]


def _combine_body(x_ref, sw_ref, p_ref, b_ref, o_ref):
    o_ref[...] = (p_ref[0] + p_ref[1] + b_ref[...]
                  + jnp.dot(x_ref[...], sw_ref[...],
                            preferred_element_type=jnp.float32, precision=_PREC))


def _sc_rgcn(n, r, d, e, hflat, src, dst, typ):
    NC, NS, L = 2, 16, 16
    RN = r * n
    CC = 80
    EW = e // (NC * NS)
    EC = e // NS
    TAB = RN // NS
    OROWS = (n // NS) // 8 * 8
    OTAIL = n - OROWS * NS
    ZB = 1024
    ztab = list(range(0, TAB - ZB, ZB)) + [TAB - ZB]
    zout = list(range(0, OROWS - CC, CC)) + [OROWS - CC]
    mesh = plsc.VectorSubcoreMesh(core_axis_name="c", subcore_axis_name="s")

    @functools.partial(
        pl.kernel,
        out_type=jax.ShapeDtypeStruct((NC, n, d), jnp.float32),
        mesh=mesh,
        scratch_types=[
            pltpu.VMEM_SHARED((RN,), jnp.float32),
            pltpu.VMEM_SHARED((n, d), jnp.float32),
            pltpu.VMEM((CC,), jnp.int32),
            pltpu.VMEM((CC,), jnp.int32),
            pltpu.VMEM((CC,), jnp.int32),
            pltpu.VMEM((CC,), jnp.int32),
            pltpu.VMEM((CC,), jnp.int32),
            pltpu.VMEM((CC,), jnp.float32),
            pltpu.VMEM((CC, d), jnp.float32),
            pltpu.VMEM((ZB,), jnp.float32),
            pltpu.VMEM((TAB + L,), jnp.float32),
            pltpu.VMEM((TAB + L,), jnp.float32),
        ],
    )
    def body(h_hbm, src_hbm, dst_hbm, typ_hbm, out_hbm,
             tab_sh, out_sh, srcb, dstb, typb, keyb, gidxb, cb, msgb,
             zb, cntv, recv):
        cid = lax.axis_index("c")
        sid = lax.axis_index("s")
        zeros = jnp.zeros((L,), jnp.float32)
        for i in range(ZB // L):
            zb[pl.ds(i * L, L)] = zeros

        def zrow(j, carry):
            for v in range(d // L):
                msgb[j, pl.ds(v * L, L)] = zeros
            return carry
        lax.fori_loop(0, CC, zrow, 0)

        tb = sid * TAB
        for s0 in ztab:
            pltpu.sync_copy(zb, tab_sh.at[pl.ds(tb + s0, ZB)])
        ob = sid * OROWS
        for s0 in zout:
            pltpu.sync_copy(msgb, out_sh.at[pl.ds(ob + s0, CC)])
        if OTAIL:
            @pl.when(sid == NS - 1)
            def _():
                pltpu.sync_copy(msgb.at[pl.ds(0, OTAIL)],
                                out_sh.at[pl.ds(NS * OROWS, OTAIL)])
        plsc.subcore_barrier()

        ones = jnp.ones((L,), jnp.float32)
        for i in range(CC // L):
            cb[pl.ds(i * L, L)] = ones

        def cbody(c, carry):
            eb = sid * EC + c * CC
            pltpu.sync_copy(typ_hbm.at[pl.ds(eb, CC)], typb)
            pltpu.sync_copy(dst_hbm.at[pl.ds(eb, CC)], dstb)
            for j in range(CC // L):
                sl = pl.ds(j * L, L)
                keyb[sl] = typb[sl] * n + dstb[sl]
            pltpu.sync_copy(cb, tab_sh.at[keyb], add=True)
            return carry
        lax.fori_loop(0, EC // CC, cbody, 0)
        plsc.subcore_barrier()

        for s0 in ztab:
            pltpu.sync_copy(tab_sh.at[pl.ds(tb + s0, ZB)], cntv.at[pl.ds(s0, ZB)])

        def rbody(i, carry):
            st = jnp.minimum(i * L, TAB - L)
            v = cntv[pl.ds(st, L)]
            recv[pl.ds(st, L)] = 1.0 / jnp.maximum(v, 1.0)
            return carry
        lax.fori_loop(0, (TAB + L - 1) // L, rbody, 0)
        for s0 in ztab:
            pltpu.sync_copy(recv.at[pl.ds(s0, ZB)], tab_sh.at[pl.ds(tb + s0, ZB)])
        plsc.subcore_barrier()

        wid = cid * NS + sid

        def mbody(c, carry):
            eb = wid * EW + c * CC
            pltpu.sync_copy(src_hbm.at[pl.ds(eb, CC)], srcb)
            pltpu.sync_copy(dst_hbm.at[pl.ds(eb, CC)], dstb)
            pltpu.sync_copy(typ_hbm.at[pl.ds(eb, CC)], typb)
            for j in range(CC // L):
                sl = pl.ds(j * L, L)
                t16 = typb[sl] * n
                keyb[sl] = t16 + dstb[sl]
                gidxb[sl] = t16 + srcb[sl]
            pltpu.sync_copy(tab_sh.at[keyb], cb)
            pltpu.sync_copy(h_hbm.at[gidxb], msgb)

            dn = lax.GatherDimensionNumbers(
                offset_dims=(), collapsed_slice_dims=(0,), start_index_map=(0,))

            def sbody(g, carry2):
                c16 = cb[pl.ds(g * L, L)]
                for jj in range(L):
                    bc = lax.gather(
                        c16, jnp.full((L, 1), jj, jnp.int32), dn,
                        slice_sizes=(1,),
                        mode=lax.GatherScatterMode.PROMISE_IN_BOUNDS)
                    j = g * L + jj
                    for v in range(d // L):
                        sl = pl.ds(v * L, L)
                        msgb[j, sl] = msgb[j, sl] * bc
                return carry2
            lax.fori_loop(0, CC // L, sbody, 0)
            pltpu.sync_copy(msgb, out_sh.at[dstb], add=True)
            return carry
        lax.fori_loop(0, EW // CC, mbody, 0)
        plsc.subcore_barrier()

        pltpu.sync_copy(out_sh.at[pl.ds(ob, OROWS)],
                        out_hbm.at[cid, pl.ds(ob, OROWS)])
        if OTAIL:
            @pl.when(sid == NS - 1)
            def _():
                pltpu.sync_copy(out_sh.at[pl.ds(NS * OROWS, OTAIL)],
                                out_hbm.at[cid, pl.ds(NS * OROWS, OTAIL)])

    return body(hflat, src, dst, typ)


def kernel(x, edge_index, edge_type, basis, att, self_weight, bias):
    n, din = x.shape
    dout = self_weight.shape[1]
    r = att.shape[0]
    b = basis.shape[0]
    e = edge_type.shape[0]
    f32 = jnp.float32

    w2 = pl.pallas_call(
        _w_body,
        out_shape=jax.ShapeDtypeStruct((r, din * dout), f32),
    )(att, basis.reshape(b, din * dout))
    w = w2.reshape(r, din, dout)

    TN = 1000
    h = pl.pallas_call(
        _h_body,
        grid=(r, n // TN),
        in_specs=[pl.BlockSpec((TN, din), lambda i, j: (j, 0)),
                  pl.BlockSpec((1, din, dout), lambda i, j: (i, 0, 0))],
        out_specs=pl.BlockSpec((1, TN, dout), lambda i, j: (i, j, 0)),
        out_shape=jax.ShapeDtypeStruct((r, n, dout), f32),
    )(x, w)

    partials = _sc_rgcn(n, r, dout, e,
                        h.reshape(r * n, dout),
                        edge_index[0], edge_index[1], edge_type)

    out = pl.pallas_call(
        _combine_body,
        grid=(n // TN,),
        in_specs=[pl.BlockSpec((TN, din), lambda j: (j, 0)),
                  pl.BlockSpec((din, dout), lambda j: (0, 0)),
                  pl.BlockSpec((2, TN, dout), lambda j: (0, j, 0)),
                  pl.BlockSpec((1, dout), lambda j: (0, 0))],
        out_specs=pl.BlockSpec((TN, dout), lambda j: (j, 0)),
        out_shape=jax.ShapeDtypeStruct((n, dout), f32),
    )(x, self_weight, partials, bias.reshape(1, dout))
    return out


# speedup vs baseline: 50.8245x; 1.5275x over previous
import functools

import jax
import jax.numpy as jnp
from jax import lax
from jax.experimental import pallas as pl
from jax.experimental.pallas import tpu as pltpu
from jax.experimental.pallas import tpu_sc as plsc

_PREC = lax.Precision.HIGHEST


def _w_body(att_ref, basis_ref, w_ref):
    w_ref[...] = lax.dot_general(
        att_ref[...], basis_ref[...], (((1,), (0,)), ((), ())),
        preferred_element_type=jnp.float32, precision=_PREC)


def _h_body(x_ref, w_ref, h_ref):
    h_ref[...] = jnp.dot(x_ref[...], w_ref[0],
                         preferred_element_type=jnp.float32,
                         precision=_PREC)[None]


def _combine_body(x_ref, sw_ref, p_ref, b_ref, o_ref):
    o_ref[...] = (jnp.concatenate([p_ref[0], p_ref[1]], axis=1) + b_ref[...]
                  + jnp.dot(x_ref[...], sw_ref[...],
                            preferred_element_type=jnp.float32, precision=_PREC))


def _sc_rgcn(n, r, d, e, hflat, src, dst, typ):
    NC, NS, L = 2, 16, 16
    RN = r * n
    DH = d // NC
    CC = 80
    NB = 5
    EC = e // NS
    NG = EC // CC // NB
    TAB = RN // NS
    OROWS = (n // NS) // 8 * 8
    OTAIL = n - OROWS * NS
    ZB = 1024
    ztab = list(range(0, TAB - ZB, ZB)) + [TAB - ZB]
    zout = list(range(0, OROWS - CC, CC)) + [OROWS - CC]
    rchunks = [(s, min(ZB, TAB - s)) for s in range(0, TAB, ZB)]
    mesh = plsc.VectorSubcoreMesh(core_axis_name="c", subcore_axis_name="s")

    @functools.partial(
        pl.kernel,
        out_type=jax.ShapeDtypeStruct((NC, n, DH), jnp.float32),
        mesh=mesh,
        compiler_params=pltpu.CompilerParams(use_tc_tiling_on_sc=False),
        scratch_types=[
            pltpu.VMEM_SHARED((RN,), jnp.float32),
            pltpu.VMEM_SHARED((n, DH), jnp.float32),
            pltpu.VMEM((NB, CC), jnp.int32),
            pltpu.VMEM((NB, CC), jnp.int32),
            pltpu.VMEM((NB, CC), jnp.int32),
            pltpu.VMEM((NB, CC), jnp.int32),
            pltpu.VMEM((NB, CC), jnp.int32),
            pltpu.VMEM((NB, CC), jnp.int32),
            pltpu.VMEM((NB, CC), jnp.float32),
            pltpu.VMEM((NB, CC, DH), jnp.float32),
            pltpu.VMEM((CC,), jnp.float32),
            pltpu.VMEM((ZB,), jnp.float32),
            pltpu.VMEM((ZB,), jnp.float32),
        ] + [pltpu.SemaphoreType.DMA] * (4 * NB),
    )
    def body(h_hbm, src_hbm, dst_hbm, typ_hbm, out_hbm,
             tab_sh, out_sh, srcb, dstb, typb, keyb, gidxb, sidxb, cbr, msgb,
             onesb, zb, rb, *sems):
        sem_l = sems[0:NB]
        sem_g = sems[NB:2 * NB]
        sem_c = sems[2 * NB:3 * NB]
        sem_s = sems[3 * NB:4 * NB]
        cid = lax.axis_index("c")
        sid = lax.axis_index("s")
        zeros = jnp.zeros((L,), jnp.float32)
        for i in range(ZB // L):
            zb[pl.ds(i * L, L)] = zeros

        def zrow(j, carry):
            for v in range(DH // L):
                msgb[0, j, pl.ds(v * L, L)] = zeros
            return carry
        lax.fori_loop(0, CC, zrow, 0)

        tb = sid * TAB
        for s0 in ztab:
            pltpu.sync_copy(zb, tab_sh.at[pl.ds(tb + s0, ZB)])
        ob = sid * OROWS
        for s0 in zout:
            pltpu.sync_copy(msgb.at[0], out_sh.at[pl.ds(ob + s0, CC)])
        if OTAIL:
            @pl.when(sid == NS - 1)
            def _():
                pltpu.sync_copy(msgb.at[0, pl.ds(0, OTAIL)],
                                out_sh.at[pl.ds(NS * OROWS, OTAIL)])
        plsc.subcore_barrier()

        ones = jnp.ones((L,), jnp.float32)
        for i in range(CC // L):
            onesb[pl.ds(i * L, L)] = ones

        def c_loads(g, b):
            eb = sid * EC + (g * NB + b) * CC
            pltpu.async_copy(typ_hbm.at[pl.ds(eb, CC)], typb.at[b], sem_l[b])
            pltpu.async_copy(dst_hbm.at[pl.ds(eb, CC)], dstb.at[b], sem_l[b])

        def c_loads_wait(b):
            pltpu.make_async_copy(typ_hbm.at[pl.ds(0, CC)], typb.at[b],
                                  sem_l[b]).wait()
            pltpu.make_async_copy(dst_hbm.at[pl.ds(0, CC)], dstb.at[b],
                                  sem_l[b]).wait()

        def c_keys(b):
            for q in range(CC // L):
                sl = pl.ds(q * L, L)
                keyb[b, sl] = typb[b, sl] * n + dstb[b, sl]

        def c_add(b):
            pltpu.async_copy(onesb, tab_sh.at[keyb.at[b]], sem_s[b], add=True)

        def c_add_wait(b):
            pltpu.make_async_copy(onesb, tab_sh.at[keyb.at[b]],
                                  sem_s[b]).wait()

        for b in range(NB):
            c_loads(0, b)
        for b in range(NB):
            c_loads_wait(b)
            c_keys(b)
            c_add(b)
            c_loads(1, b)

        def cgroup(g, carry):
            for b in range(NB):
                c_loads_wait(b)
                c_add_wait(b)
                c_keys(b)
                c_add(b)

                @pl.when(g < NG - 1)
                def _():
                    c_loads(g + 1, b)
            return carry
        lax.fori_loop(1, NG, cgroup, 0)
        for b in range(NB):
            c_add_wait(b)
        plsc.subcore_barrier()

        for s0, ln in rchunks:
            pltpu.sync_copy(tab_sh.at[pl.ds(tb + s0, ln)], zb.at[pl.ds(0, ln)])

            def rbody(i, carry, ln=ln):
                st = jnp.minimum(i * L, ln - L)
                v = zb[pl.ds(st, L)]
                rb[pl.ds(st, L)] = 1.0 / jnp.maximum(v, 1.0)
                return carry
            lax.fori_loop(0, (ln + L - 1) // L, rbody, 0)
            pltpu.sync_copy(rb.at[pl.ds(0, ln)], tab_sh.at[pl.ds(tb + s0, ln)])
        plsc.subcore_barrier()

        dn = lax.GatherDimensionNumbers(
            offset_dims=(), collapsed_slice_dims=(0,), start_index_map=(0,))

        def m_loads(g, b):
            eb = sid * EC + (g * NB + b) * CC
            pltpu.async_copy(src_hbm.at[pl.ds(eb, CC)], srcb.at[b], sem_l[b])
            pltpu.async_copy(dst_hbm.at[pl.ds(eb, CC)], dstb.at[b], sem_l[b])
            pltpu.async_copy(typ_hbm.at[pl.ds(eb, CC)], typb.at[b], sem_l[b])

        def m_loads_wait(b):
            for ref in (srcb, dstb, typb):
                pltpu.make_async_copy(src_hbm.at[pl.ds(0, CC)], ref.at[b],
                                      sem_l[b]).wait()

        def m_keys(b):
            for q in range(CC // L):
                sl = pl.ds(q * L, L)
                t16 = typb[b, sl] * n
                keyb[b, sl] = t16 + dstb[b, sl]
                gidxb[b, sl] = (t16 + srcb[b, sl]) * 2 + cid
                sidxb[b, sl] = dstb[b, sl]

        def m_gathers(b):
            pltpu.async_copy(tab_sh.at[keyb.at[b]], cbr.at[b], sem_c[b])
            pltpu.async_copy(h_hbm.at[gidxb.at[b]], msgb.at[b], sem_g[b])

        def m_gathers_wait(b):
            pltpu.make_async_copy(tab_sh.at[keyb.at[b]], cbr.at[b],
                                  sem_c[b]).wait()
            pltpu.make_async_copy(h_hbm.at[gidxb.at[b]], msgb.at[b],
                                  sem_g[b]).wait()

        def m_scale(b):
            def sbody(q, carry2):
                c16 = cbr[b, pl.ds(q * L, L)]
                for jj in range(L):
                    bc = lax.gather(
                        c16, jnp.full((L, 1), jj, jnp.int32), dn,
                        slice_sizes=(1,),
                        mode=lax.GatherScatterMode.PROMISE_IN_BOUNDS)
                    for v in range(DH // L):
                        sl = pl.ds(v * L, L)
                        msgb[b, q * L + jj, sl] = msgb[b, q * L + jj, sl] * bc
                return carry2
            lax.fori_loop(0, CC // L, sbody, 0)

        def m_scatter(b):
            pltpu.async_copy(msgb.at[b], out_sh.at[sidxb.at[b]], sem_s[b],
                             add=True)

        def m_scatter_wait(b):
            pltpu.make_async_copy(msgb.at[b], out_sh.at[sidxb.at[b]],
                                  sem_s[b]).wait()

        for b in range(NB):
            m_loads(0, b)
        for b in range(NB):
            m_loads_wait(b)
            m_keys(b)
            m_gathers(b)

        def mgroup(g, carry):
            for b in range(NB):
                m_loads(g, b)
            for b in range(NB):
                m_gathers_wait(b)
                m_scale(b)
                m_scatter(b)
            for b in range(NB):
                m_loads_wait(b)
                m_scatter_wait(b)
                m_keys(b)
                m_gathers(b)
            return carry
        lax.fori_loop(1, NG, mgroup, 0)
        for b in range(NB):
            m_gathers_wait(b)
            m_scale(b)
            m_scatter(b)
        for b in range(NB):
            m_scatter_wait(b)
        plsc.subcore_barrier()

        pltpu.sync_copy(out_sh.at[pl.ds(ob, OROWS)],
                        out_hbm.at[cid, pl.ds(ob, OROWS)])
        if OTAIL:
            @pl.when(sid == NS - 1)
            def _():
                pltpu.sync_copy(out_sh.at[pl.ds(NS * OROWS, OTAIL)],
                                out_hbm.at[cid, pl.ds(NS * OROWS, OTAIL)])

    return body(hflat, src, dst, typ)


def kernel(x, edge_index, edge_type, basis, att, self_weight, bias):
    n, din = x.shape
    dout = self_weight.shape[1]
    r = att.shape[0]
    b = basis.shape[0]
    e = edge_type.shape[0]
    dh = dout // 2
    f32 = jnp.float32

    w2 = pl.pallas_call(
        _w_body,
        out_shape=jax.ShapeDtypeStruct((r, din * dout), f32),
    )(att, basis.reshape(b, din * dout))
    w = w2.reshape(r, din, dout)

    TN = 1000
    h = pl.pallas_call(
        _h_body,
        grid=(r, n // TN),
        in_specs=[pl.BlockSpec((TN, din), lambda i, j: (j, 0)),
                  pl.BlockSpec((1, din, dout), lambda i, j: (i, 0, 0))],
        out_specs=pl.BlockSpec((1, TN, dout), lambda i, j: (i, j, 0)),
        out_shape=jax.ShapeDtypeStruct((r, n, dout), f32),
    )(x, w)

    partials = _sc_rgcn(n, r, dout, e,
                        h.reshape(2 * r * n, dh),
                        edge_index[0], edge_index[1], edge_type)

    out = pl.pallas_call(
        _combine_body,
        grid=(n // TN,),
        in_specs=[pl.BlockSpec((TN, din), lambda j: (j, 0)),
                  pl.BlockSpec((din, dout), lambda j: (0, 0)),
                  pl.BlockSpec((2, TN, dh), lambda j: (0, j, 0)),
                  pl.BlockSpec((1, dout), lambda j: (0, 0))],
        out_specs=pl.BlockSpec((TN, dout), lambda j: (j, 0)),
        out_shape=jax.ShapeDtypeStruct((n, dout), f32),
    )(x, self_weight, partials, bias.reshape(1, dout))
    return out


# speedup vs baseline: 53.9681x; 1.0619x over previous
import functools

import jax
import jax.numpy as jnp
from jax import lax
from jax.experimental import pallas as pl
from jax.experimental.pallas import tpu as pltpu
from jax.experimental.pallas import tpu_sc as plsc

_PREC = lax.Precision.DEFAULT


def _w_body(att_ref, basis_ref, w_ref):
    w_ref[...] = lax.dot_general(
        att_ref[...], basis_ref[...], (((1,), (0,)), ((), ())),
        preferred_element_type=jnp.float32, precision=_PREC)


def _h_body(x_ref, w_ref, h_ref):
    h_ref[...] = jnp.dot(x_ref[...], w_ref[0],
                         preferred_element_type=jnp.float32,
                         precision=_PREC)[None]


def _combine_body(x_ref, sw_ref, p_ref, b_ref, o_ref):
    o_ref[...] = (jnp.concatenate([p_ref[0], p_ref[1]], axis=1) + b_ref[...]
                  + jnp.dot(x_ref[...], sw_ref[...],
                            preferred_element_type=jnp.float32, precision=_PREC))


def _sc_rgcn(n, r, d, e, hflat, src, dst, typ):
    NC, NS, L = 2, 16, 16
    RN = r * n
    DH = d // NC
    CC = 80
    NB = 5
    EC = e // NS
    NG = EC // CC // NB
    TAB = RN // NS
    OROWS = (n // NS) // 8 * 8
    OTAIL = n - OROWS * NS
    ZB = 1024
    ztab = list(range(0, TAB - ZB, ZB)) + [TAB - ZB]
    zout = list(range(0, OROWS - CC, CC)) + [OROWS - CC]
    rchunks = [(s, min(ZB, TAB - s)) for s in range(0, TAB, ZB)]
    mesh = plsc.VectorSubcoreMesh(core_axis_name="c", subcore_axis_name="s")

    @functools.partial(
        pl.kernel,
        out_type=jax.ShapeDtypeStruct((NC, n, DH), jnp.float32),
        mesh=mesh,
        compiler_params=pltpu.CompilerParams(use_tc_tiling_on_sc=False),
        scratch_types=[
            pltpu.VMEM_SHARED((RN,), jnp.float32),
            pltpu.VMEM_SHARED((n, DH), jnp.float32),
            pltpu.VMEM((NB, CC), jnp.int32),
            pltpu.VMEM((NB, CC), jnp.int32),
            pltpu.VMEM((NB, CC), jnp.int32),
            pltpu.VMEM((NB, CC), jnp.int32),
            pltpu.VMEM((NB, CC), jnp.int32),
            pltpu.VMEM((NB, CC), jnp.int32),
            pltpu.VMEM((NB, CC), jnp.float32),
            pltpu.VMEM((NB, CC, DH), jnp.float32),
            pltpu.VMEM((CC,), jnp.float32),
            pltpu.VMEM((ZB,), jnp.float32),
            pltpu.VMEM((ZB,), jnp.float32),
        ] + [pltpu.SemaphoreType.DMA] * (4 * NB),
    )
    def body(h_hbm, src_hbm, dst_hbm, typ_hbm, out_hbm,
             tab_sh, out_sh, srcb, dstb, typb, keyb, gidxb, sidxb, cbr, msgb,
             onesb, zb, rb, *sems):
        sem_l = sems[0:NB]
        sem_g = sems[NB:2 * NB]
        sem_c = sems[2 * NB:3 * NB]
        sem_s = sems[3 * NB:4 * NB]
        cid = lax.axis_index("c")
        sid = lax.axis_index("s")
        zeros = jnp.zeros((L,), jnp.float32)
        for i in range(ZB // L):
            zb[pl.ds(i * L, L)] = zeros

        def zrow(j, carry):
            for v in range(DH // L):
                msgb[0, j, pl.ds(v * L, L)] = zeros
            return carry
        lax.fori_loop(0, CC, zrow, 0)

        tb = sid * TAB
        for s0 in ztab:
            pltpu.sync_copy(zb, tab_sh.at[pl.ds(tb + s0, ZB)])
        ob = sid * OROWS
        for s0 in zout:
            pltpu.sync_copy(msgb.at[0], out_sh.at[pl.ds(ob + s0, CC)])
        if OTAIL:
            @pl.when(sid == NS - 1)
            def _():
                pltpu.sync_copy(msgb.at[0, pl.ds(0, OTAIL)],
                                out_sh.at[pl.ds(NS * OROWS, OTAIL)])
        plsc.subcore_barrier()

        ones = jnp.ones((L,), jnp.float32)
        for i in range(CC // L):
            onesb[pl.ds(i * L, L)] = ones

        def c_loads(g, b):
            eb = sid * EC + (g * NB + b) * CC
            pltpu.async_copy(typ_hbm.at[pl.ds(eb, CC)], typb.at[b], sem_l[b])
            pltpu.async_copy(dst_hbm.at[pl.ds(eb, CC)], dstb.at[b], sem_l[b])

        def c_loads_wait(b):
            pltpu.make_async_copy(typ_hbm.at[pl.ds(0, CC)], typb.at[b],
                                  sem_l[b]).wait()
            pltpu.make_async_copy(dst_hbm.at[pl.ds(0, CC)], dstb.at[b],
                                  sem_l[b]).wait()

        def c_keys(b):
            for q in range(CC // L):
                sl = pl.ds(q * L, L)
                keyb[b, sl] = typb[b, sl] * n + dstb[b, sl]

        def c_add(b):
            pltpu.async_copy(onesb, tab_sh.at[keyb.at[b]], sem_s[b], add=True)

        def c_add_wait(b):
            pltpu.make_async_copy(onesb, tab_sh.at[keyb.at[b]],
                                  sem_s[b]).wait()

        _ABL = NB
        for b in range(_ABL):
            c_loads(0, b)
        for b in range(_ABL):
            c_loads_wait(b)
            c_keys(b)
            c_add(b)
            c_loads(1, b)

        def cgroup(g, carry):
            for b in range(NB):
                c_loads_wait(b)
                c_add_wait(b)
                c_keys(b)
                c_add(b)

                @pl.when(g < NG - 1)
                def _():
                    c_loads(g + 1, b)
            return carry
        lax.fori_loop(1, 1 if not _ABL else NG, cgroup, 0)
        for b in range(_ABL):
            c_add_wait(b)
        plsc.subcore_barrier()

        for s0, ln in (rchunks if _ABL else []):
            pltpu.sync_copy(tab_sh.at[pl.ds(tb + s0, ln)], zb.at[pl.ds(0, ln)])

            def rbody(i, carry, ln=ln):
                st = jnp.minimum(i * L, ln - L)
                v = zb[pl.ds(st, L)]
                rb[pl.ds(st, L)] = 1.0 / jnp.maximum(v, 1.0)
                return carry
            lax.fori_loop(0, (ln + L - 1) // L, rbody, 0)
            pltpu.sync_copy(rb.at[pl.ds(0, ln)], tab_sh.at[pl.ds(tb + s0, ln)])
        plsc.subcore_barrier()

        dn = lax.GatherDimensionNumbers(
            offset_dims=(), collapsed_slice_dims=(0,), start_index_map=(0,))

        def m_loads(g, b):
            eb = sid * EC + (g * NB + b) * CC
            pltpu.async_copy(src_hbm.at[pl.ds(eb, CC)], srcb.at[b], sem_l[b])
            pltpu.async_copy(dst_hbm.at[pl.ds(eb, CC)], dstb.at[b], sem_l[b])
            pltpu.async_copy(typ_hbm.at[pl.ds(eb, CC)], typb.at[b], sem_l[b])

        def m_loads_wait(b):
            for ref in (srcb, dstb, typb):
                pltpu.make_async_copy(src_hbm.at[pl.ds(0, CC)], ref.at[b],
                                      sem_l[b]).wait()

        def m_keys(b):
            for q in range(CC // L):
                sl = pl.ds(q * L, L)
                t16 = typb[b, sl] * n
                keyb[b, sl] = t16 + dstb[b, sl]
                gidxb[b, sl] = (t16 + srcb[b, sl]) * 2 + cid
                sidxb[b, sl] = dstb[b, sl]

        def m_gathers(b):
            pltpu.async_copy(tab_sh.at[keyb.at[b]], cbr.at[b], sem_c[b])
            pltpu.async_copy(h_hbm.at[gidxb.at[b]], msgb.at[b], sem_g[b])

        def m_gathers_wait(b):
            pltpu.make_async_copy(tab_sh.at[keyb.at[b]], cbr.at[b],
                                  sem_c[b]).wait()
            pltpu.make_async_copy(h_hbm.at[gidxb.at[b]], msgb.at[b],
                                  sem_g[b]).wait()

        def m_scale(b):
            def sbody(q, carry2):
                c16 = cbr[b, pl.ds(q * L, L)]
                for jj in range(L):
                    bc = lax.gather(
                        c16, jnp.full((L, 1), jj, jnp.int32), dn,
                        slice_sizes=(1,),
                        mode=lax.GatherScatterMode.PROMISE_IN_BOUNDS)
                    for v in range(DH // L):
                        sl = pl.ds(v * L, L)
                        msgb[b, q * L + jj, sl] = msgb[b, q * L + jj, sl] * bc
                return carry2
            lax.fori_loop(0, CC // L, sbody, 0)

        def m_scatter(b):
            pltpu.async_copy(msgb.at[b], out_sh.at[sidxb.at[b]], sem_s[b],
                             add=True)

        def m_scatter_wait(b):
            pltpu.make_async_copy(msgb.at[b], out_sh.at[sidxb.at[b]],
                                  sem_s[b]).wait()

        _ABLM = NB
        for b in range(_ABLM):
            m_loads(0, b)
        for b in range(_ABLM):
            m_loads_wait(b)
            m_keys(b)
            m_gathers(b)

        def mgroup(g, carry):
            for b in range(NB):
                m_loads(g, b)
            for b in range(NB):
                m_gathers_wait(b)
                m_scale(b)
                m_scatter(b)
            for b in range(NB):
                m_loads_wait(b)
                m_scatter_wait(b)
                m_keys(b)
                m_gathers(b)
            return carry
        lax.fori_loop(1, 1 if not _ABLM else NG, mgroup, 0)
        for b in range(_ABLM):
            m_gathers_wait(b)
            m_scale(b)
            m_scatter(b)
        for b in range(_ABLM):
            m_scatter_wait(b)
        plsc.subcore_barrier()

        pltpu.sync_copy(out_sh.at[pl.ds(ob, OROWS)],
                        out_hbm.at[cid, pl.ds(ob, OROWS)])
        if OTAIL:
            @pl.when(sid == NS - 1)
            def _():
                pltpu.sync_copy(out_sh.at[pl.ds(NS * OROWS, OTAIL)],
                                out_hbm.at[cid, pl.ds(NS * OROWS, OTAIL)])

    return body(hflat, src, dst, typ)


def kernel(x, edge_index, edge_type, basis, att, self_weight, bias):
    n, din = x.shape
    dout = self_weight.shape[1]
    r = att.shape[0]
    b = basis.shape[0]
    e = edge_type.shape[0]
    dh = dout // 2
    f32 = jnp.float32

    w2 = pl.pallas_call(
        _w_body,
        out_shape=jax.ShapeDtypeStruct((r, din * dout), f32),
    )(att, basis.reshape(b, din * dout))
    w = w2.reshape(r, din, dout)

    TN = 1000
    h = pl.pallas_call(
        _h_body,
        grid=(r, n // TN),
        in_specs=[pl.BlockSpec((TN, din), lambda i, j: (j, 0)),
                  pl.BlockSpec((1, din, dout), lambda i, j: (i, 0, 0))],
        out_specs=pl.BlockSpec((1, TN, dout), lambda i, j: (i, j, 0)),
        out_shape=jax.ShapeDtypeStruct((r, n, dout), f32),
    )(x, w)

    partials = _sc_rgcn(n, r, dout, e,
                        h.reshape(2 * r * n, dh),
                        edge_index[0], edge_index[1], edge_type)

    out = pl.pallas_call(
        _combine_body,
        grid=(n // TN,),
        in_specs=[pl.BlockSpec((TN, din), lambda j: (j, 0)),
                  pl.BlockSpec((din, dout), lambda j: (0, 0)),
                  pl.BlockSpec((2, TN, dh), lambda j: (0, j, 0)),
                  pl.BlockSpec((1, dout), lambda j: (0, 0))],
        out_specs=pl.BlockSpec((TN, dout), lambda j: (j, 0)),
        out_shape=jax.ShapeDtypeStruct((n, dout), f32),
    )(x, self_weight, partials, bias.reshape(1, dout))
    return out


# speedup vs baseline: 71.4000x; 1.3230x over previous
import functools

import jax
import jax.numpy as jnp
from jax import lax
from jax.experimental import pallas as pl
from jax.experimental.pallas import tpu as pltpu
from jax.experimental.pallas import tpu_sc as plsc

_PREC = lax.Precision.DEFAULT


def _w_body(att_ref, basis_ref, w_ref):
    w_ref[...] = lax.dot_general(
        att_ref[...], basis_ref[...], (((1,), (0,)), ((), ())),
        preferred_element_type=jnp.float32, precision=_PREC)


def _h_body(x_ref, w_ref, h_ref):
    h_ref[...] = jnp.dot(x_ref[...], w_ref[0],
                         preferred_element_type=jnp.float32,
                         precision=_PREC)[None]


def _combine_body(x_ref, sw_ref, p_ref, b_ref, o_ref):
    o_ref[...] = (jnp.concatenate([p_ref[0], p_ref[1]], axis=1) + b_ref[...]
                  + jnp.dot(x_ref[...], sw_ref[...],
                            preferred_element_type=jnp.float32, precision=_PREC))


def _sc_rgcn(n, r, d, e, hflat, src, dst, typ):
    NC, NS, L = 2, 16, 16
    RN = r * n
    DH = d // NC
    CC = 80
    NB = 5
    EC = e // NS
    NG = EC // CC // NB
    TAB = RN // NS
    OROWS = (n // NS) // 8 * 8
    OTAIL = n - OROWS * NS
    ZB = 1024
    ztab = list(range(0, TAB - ZB, ZB)) + [TAB - ZB]
    zout = list(range(0, OROWS - CC, CC)) + [OROWS - CC]
    rchunks = [(s, min(ZB, TAB - s)) for s in range(0, TAB, ZB)]
    mesh = plsc.VectorSubcoreMesh(core_axis_name="c", subcore_axis_name="s")

    @functools.partial(
        pl.kernel,
        out_type=jax.ShapeDtypeStruct((NC, n, DH), jnp.float32),
        mesh=mesh,
        compiler_params=pltpu.CompilerParams(use_tc_tiling_on_sc=False),
        scratch_types=[
            pltpu.VMEM_SHARED((RN,), jnp.float32),
            pltpu.VMEM_SHARED((n, DH), jnp.float32),
            pltpu.VMEM((2 * NB, CC), jnp.int32),
            pltpu.VMEM((2 * NB, CC), jnp.int32),
            pltpu.VMEM((2 * NB, CC), jnp.int32),
            pltpu.VMEM((2 * NB, CC), jnp.int32),
            pltpu.VMEM((2 * NB, CC), jnp.int32),
            pltpu.VMEM((2 * NB, CC), jnp.int32),
            pltpu.VMEM((2 * NB, CC), jnp.float32),
            pltpu.VMEM((2 * NB, CC, DH), jnp.float32),
            pltpu.VMEM((CC,), jnp.float32),
            pltpu.VMEM((ZB,), jnp.float32),
            pltpu.VMEM((ZB,), jnp.float32),
        ] + [pltpu.SemaphoreType.DMA] * 8,
    )
    def body(h_hbm, src_hbm, dst_hbm, typ_hbm, out_hbm,
             tab_sh, out_sh, srcb, dstb, typb, keyb, gidxb, sidxb, cbr, msgb,
             onesb, zb, rb, *sems):
        sem_l = sems[0:2]
        sem_g = sems[2:4]
        sem_c = sems[4:6]
        sem_s = sems[6:8]
        cid = lax.axis_index("c")
        sid = lax.axis_index("s")
        zeros = jnp.zeros((L,), jnp.float32)
        for i in range(ZB // L):
            zb[pl.ds(i * L, L)] = zeros

        def zrow(j, carry):
            for v in range(DH // L):
                msgb[0, j, pl.ds(v * L, L)] = zeros
            return carry
        lax.fori_loop(0, CC, zrow, 0)

        tb = sid * TAB
        for s0 in ztab:
            pltpu.sync_copy(zb, tab_sh.at[pl.ds(tb + s0, ZB)])
        ob = sid * OROWS
        for s0 in zout:
            pltpu.sync_copy(msgb.at[0], out_sh.at[pl.ds(ob + s0, CC)])
        if OTAIL:
            @pl.when(sid == NS - 1)
            def _():
                pltpu.sync_copy(msgb.at[0, pl.ds(0, OTAIL)],
                                out_sh.at[pl.ds(NS * OROWS, OTAIL)])
        plsc.subcore_barrier()

        ones = jnp.ones((L,), jnp.float32)
        for i in range(CC // L):
            onesb[pl.ds(i * L, L)] = ones

        def c_loads(g):
            for b in range(NB):
                eb = sid * EC + (g * NB + b) * CC
                pltpu.async_copy(typ_hbm.at[pl.ds(eb, CC)], typb.at[b],
                                 sem_l[0])
                pltpu.async_copy(dst_hbm.at[pl.ds(eb, CC)], dstb.at[b],
                                 sem_l[0])

        def c_loads_wait():
            for b in range(NB):
                pltpu.make_async_copy(typ_hbm.at[pl.ds(0, CC)], typb.at[b],
                                      sem_l[0]).wait()
                pltpu.make_async_copy(dst_hbm.at[pl.ds(0, CC)], dstb.at[b],
                                      sem_l[0]).wait()

        def c_keys(b):
            for q in range(CC // L):
                sl = pl.ds(q * L, L)
                keyb[b, sl] = typb[b, sl] * n + dstb[b, sl]

        def c_add(b):
            pltpu.async_copy(onesb, tab_sh.at[keyb.at[b]], sem_s[0], add=True)

        def c_adds_wait():
            for b in range(NB):
                pltpu.make_async_copy(onesb, tab_sh.at[keyb.at[b]],
                                      sem_s[0]).wait()

        c_loads(0)
        c_loads_wait()
        for b in range(NB):
            c_keys(b)
            c_add(b)
        c_loads(1)

        def cgroup(g, carry):
            c_loads_wait()
            c_adds_wait()
            for b in range(NB):
                c_keys(b)
                c_add(b)

            @pl.when(g < NG - 1)
            def _():
                c_loads(g + 1)
            return carry
        lax.fori_loop(1, NG, cgroup, 0)
        c_adds_wait()
        plsc.subcore_barrier()

        for s0, ln in rchunks:
            pltpu.sync_copy(tab_sh.at[pl.ds(tb + s0, ln)], zb.at[pl.ds(0, ln)])

            def rbody(i, carry, ln=ln):
                st = jnp.minimum(i * L, ln - L)
                v = zb[pl.ds(st, L)]
                rb[pl.ds(st, L)] = 1.0 / jnp.maximum(v, 1.0)
                return carry
            lax.fori_loop(0, (ln + L - 1) // L, rbody, 0)
            pltpu.sync_copy(rb.at[pl.ds(0, ln)], tab_sh.at[pl.ds(tb + s0, ln)])
        plsc.subcore_barrier()

        dn = lax.GatherDimensionNumbers(
            offset_dims=(), collapsed_slice_dims=(0,), start_index_map=(0,))

        def m_group_loads(g, p):
            for b in range(NB):
                s = p * NB + b
                eb = sid * EC + (g * NB + b) * CC
                pltpu.async_copy(src_hbm.at[pl.ds(eb, CC)], srcb.at[s],
                                 sem_l[p])
                pltpu.async_copy(dst_hbm.at[pl.ds(eb, CC)], dstb.at[s],
                                 sem_l[p])
                pltpu.async_copy(typ_hbm.at[pl.ds(eb, CC)], typb.at[s],
                                 sem_l[p])

        def m_group_loads_wait(p):
            for b in range(NB):
                s = p * NB + b
                for ref in (srcb, dstb, typb):
                    pltpu.make_async_copy(src_hbm.at[pl.ds(0, CC)], ref.at[s],
                                          sem_l[p]).wait()

        def m_keys(b):
            for q in range(CC // L):
                sl = pl.ds(q * L, L)
                t16 = typb[b, sl] * n
                keyb[b, sl] = t16 + dstb[b, sl]
                gidxb[b, sl] = (t16 + srcb[b, sl]) * 2 + cid
                sidxb[b, sl] = dstb[b, sl]

        def m_gathers(b, p):
            pltpu.async_copy(tab_sh.at[keyb.at[b]], cbr.at[b], sem_c[p])
            pltpu.async_copy(h_hbm.at[gidxb.at[b]], msgb.at[b], sem_g[p])

        def m_group_gathers_wait(p):
            for b in range(NB):
                s = p * NB + b
                pltpu.make_async_copy(tab_sh.at[keyb.at[s]], cbr.at[s],
                                      sem_c[p]).wait()
                pltpu.make_async_copy(h_hbm.at[gidxb.at[s]], msgb.at[s],
                                      sem_g[p]).wait()

        def m_scale(b):
            def sbody(q, carry2):
                c16 = cbr[b, pl.ds(q * L, L)]
                for jj in range(L):
                    bc = lax.gather(
                        c16, jnp.full((L, 1), jj, jnp.int32), dn,
                        slice_sizes=(1,),
                        mode=lax.GatherScatterMode.PROMISE_IN_BOUNDS)
                    for v in range(DH // L):
                        sl = pl.ds(v * L, L)
                        msgb[b, q * L + jj, sl] = msgb[b, q * L + jj, sl] * bc
                return carry2
            lax.fori_loop(0, CC // L, sbody, 0)

        def m_scatter(b, p):
            pltpu.async_copy(msgb.at[b], out_sh.at[sidxb.at[b]], sem_s[p],
                             add=True)

        def m_group_scatters_wait(p):
            for b in range(NB):
                s = p * NB + b
                pltpu.make_async_copy(msgb.at[s], out_sh.at[sidxb.at[s]],
                                      sem_s[p]).wait()

        def m_stage_in(g, p, first=False):
            m_group_loads_wait(p)
            if not first:
                m_group_scatters_wait(p)
            for b in range(NB):
                s = p * NB + b
                m_keys(s)
                m_gathers(s, p)

        def m_stage_out(p):
            m_group_gathers_wait(p)
            for b in range(NB):
                s = p * NB + b
                m_scale(s)
                m_scatter(s, p)

        m_group_loads(0, 0)
        m_stage_in(0, 0, first=True)
        m_group_loads(1, 1)
        m_stage_in(1, 1, first=True)

        def mpair(gi, carry):
            g0 = 2 * gi
            m_group_loads(g0, 0)
            m_stage_out(0)
            m_stage_in(g0, 0)
            m_group_loads(g0 + 1, 1)
            m_stage_out(1)
            m_stage_in(g0 + 1, 1)
            return carry
        lax.fori_loop(1, NG // 2, mpair, 0)
        m_stage_out(0)
        m_stage_out(1)
        m_group_scatters_wait(0)
        m_group_scatters_wait(1)
        plsc.subcore_barrier()

        pltpu.sync_copy(out_sh.at[pl.ds(ob, OROWS)],
                        out_hbm.at[cid, pl.ds(ob, OROWS)])
        if OTAIL:
            @pl.when(sid == NS - 1)
            def _():
                pltpu.sync_copy(out_sh.at[pl.ds(NS * OROWS, OTAIL)],
                                out_hbm.at[cid, pl.ds(NS * OROWS, OTAIL)])

    return body(hflat, src, dst, typ)


def kernel(x, edge_index, edge_type, basis, att, self_weight, bias):
    n, din = x.shape
    dout = self_weight.shape[1]
    r = att.shape[0]
    b = basis.shape[0]
    e = edge_type.shape[0]
    dh = dout // 2
    f32 = jnp.float32

    w2 = pl.pallas_call(
        _w_body,
        out_shape=jax.ShapeDtypeStruct((r, din * dout), f32),
    )(att, basis.reshape(b, din * dout))
    w = w2.reshape(r, din, dout)

    TN = 1000
    h = pl.pallas_call(
        _h_body,
        grid=(r, n // TN),
        in_specs=[pl.BlockSpec((TN, din), lambda i, j: (j, 0)),
                  pl.BlockSpec((1, din, dout), lambda i, j: (i, 0, 0))],
        out_specs=pl.BlockSpec((1, TN, dout), lambda i, j: (i, j, 0)),
        out_shape=jax.ShapeDtypeStruct((r, n, dout), f32),
    )(x, w)

    partials = _sc_rgcn(n, r, dout, e,
                        h.reshape(2 * r * n, dh),
                        edge_index[0], edge_index[1], edge_type)

    out = pl.pallas_call(
        _combine_body,
        grid=(n // TN,),
        in_specs=[pl.BlockSpec((TN, din), lambda j: (j, 0)),
                  pl.BlockSpec((din, dout), lambda j: (0, 0)),
                  pl.BlockSpec((2, TN, dh), lambda j: (0, j, 0)),
                  pl.BlockSpec((1, dout), lambda j: (0, 0))],
        out_specs=pl.BlockSpec((TN, dout), lambda j: (j, 0)),
        out_shape=jax.ShapeDtypeStruct((n, dout), f32),
    )(x, self_weight, partials, bias.reshape(1, dout))
    return out


# speedup vs baseline: 75.8647x; 1.0625x over previous
import functools

import jax
import jax.numpy as jnp
from jax import lax
from jax.experimental import pallas as pl
from jax.experimental.pallas import tpu as pltpu
from jax.experimental.pallas import tpu_sc as plsc

_PREC = lax.Precision.DEFAULT


def _w_body(att_ref, basis_ref, w_ref):
    w_ref[...] = lax.dot_general(
        att_ref[...], basis_ref[...], (((1,), (0,)), ((), ())),
        preferred_element_type=jnp.float32, precision=_PREC)


def _h_body(x_ref, w_ref, h_ref):
    h_ref[...] = jnp.dot(x_ref[...], w_ref[...],
                         preferred_element_type=jnp.float32,
                         precision=_PREC)


def _combine_body(x_ref, sw_ref, p_ref, b_ref, o_ref):
    o_ref[...] = (jnp.concatenate([p_ref[0], p_ref[1]], axis=1) + b_ref[...]
                  + jnp.dot(x_ref[...], sw_ref[...],
                            preferred_element_type=jnp.float32, precision=_PREC))


def _sc_rgcn(n, r, d, e, hflat, src, dst, typ):
    NC, NS, L = 2, 16, 16
    RN = r * n
    DH = d // NC
    CC = 80
    NB = 5
    EC = e // NS
    NG = EC // CC // NB
    TAB = RN // NS
    OROWS = (n // NS) // 8 * 8
    OTAIL = n - OROWS * NS
    ZB = 1024
    ztab = list(range(0, TAB - ZB, ZB)) + [TAB - ZB]
    zout = list(range(0, OROWS - CC, CC)) + [OROWS - CC]
    rchunks = [(s, min(ZB, TAB - s)) for s in range(0, TAB, ZB)]
    mesh = plsc.VectorSubcoreMesh(core_axis_name="c", subcore_axis_name="s")

    @functools.partial(
        pl.kernel,
        out_type=jax.ShapeDtypeStruct((NC, n, DH), jnp.float32),
        mesh=mesh,
        compiler_params=pltpu.CompilerParams(use_tc_tiling_on_sc=False),
        scratch_types=[
            pltpu.VMEM_SHARED((RN,), jnp.float32),
            pltpu.VMEM_SHARED((n, DH), jnp.float32),
            pltpu.VMEM((2 * NB, CC), jnp.int32),
            pltpu.VMEM((2 * NB, CC), jnp.int32),
            pltpu.VMEM((2 * NB, CC), jnp.int32),
            pltpu.VMEM((2 * NB, CC), jnp.int32),
            pltpu.VMEM((2 * NB, CC), jnp.int32),
            pltpu.VMEM((2 * NB, CC), jnp.int32),
            pltpu.VMEM((2 * NB, CC), jnp.float32),
            pltpu.VMEM((2 * NB, CC, DH), jnp.float32),
            pltpu.VMEM((CC,), jnp.float32),
            pltpu.VMEM((ZB,), jnp.float32),
            pltpu.VMEM((ZB,), jnp.float32),
        ] + [pltpu.SemaphoreType.DMA] * 8,
    )
    def body(h_hbm, src_hbm, dst_hbm, typ_hbm, out_hbm,
             tab_sh, out_sh, srcb, dstb, typb, keyb, gidxb, sidxb, cbr, msgb,
             onesb, zb, rb, *sems):
        sem_l = sems[0:2]
        sem_g = sems[2:4]
        sem_c = sems[4:6]
        sem_s = sems[6:8]
        cid = lax.axis_index("c")
        sid = lax.axis_index("s")
        zeros = jnp.zeros((L,), jnp.float32)
        for i in range(ZB // L):
            zb[pl.ds(i * L, L)] = zeros

        def zrow(j, carry):
            for v in range(DH // L):
                msgb[0, j, pl.ds(v * L, L)] = zeros
            return carry
        lax.fori_loop(0, CC, zrow, 0)

        tb = sid * TAB
        for s0 in ztab:
            pltpu.sync_copy(zb, tab_sh.at[pl.ds(tb + s0, ZB)])
        ob = sid * OROWS
        for s0 in zout:
            pltpu.sync_copy(msgb.at[0], out_sh.at[pl.ds(ob + s0, CC)])
        if OTAIL:
            @pl.when(sid == NS - 1)
            def _():
                pltpu.sync_copy(msgb.at[0, pl.ds(0, OTAIL)],
                                out_sh.at[pl.ds(NS * OROWS, OTAIL)])
        plsc.subcore_barrier()

        ones = jnp.ones((L,), jnp.float32)
        for i in range(CC // L):
            onesb[pl.ds(i * L, L)] = ones

        def c_loads(g):
            for b in range(NB):
                eb = sid * EC + (g * NB + b) * CC
                pltpu.async_copy(typ_hbm.at[pl.ds(eb, CC)], typb.at[b],
                                 sem_l[0])
                pltpu.async_copy(dst_hbm.at[pl.ds(eb, CC)], dstb.at[b],
                                 sem_l[0])

        def c_loads_wait():
            for b in range(NB):
                pltpu.make_async_copy(typ_hbm.at[pl.ds(0, CC)], typb.at[b],
                                      sem_l[0]).wait()
                pltpu.make_async_copy(dst_hbm.at[pl.ds(0, CC)], dstb.at[b],
                                      sem_l[0]).wait()

        def c_keys(b):
            for q in range(CC // L):
                sl = pl.ds(q * L, L)
                keyb[b, sl] = typb[b, sl] * n + dstb[b, sl]

        def c_add(b):
            pltpu.async_copy(onesb, tab_sh.at[keyb.at[b]], sem_s[0], add=True)

        def c_adds_wait():
            for b in range(NB):
                pltpu.make_async_copy(onesb, tab_sh.at[keyb.at[b]],
                                      sem_s[0]).wait()

        c_loads(0)
        c_loads_wait()
        for b in range(NB):
            c_keys(b)
            c_add(b)
        c_loads(1)

        def cgroup(g, carry):
            c_loads_wait()
            c_adds_wait()
            for b in range(NB):
                c_keys(b)
                c_add(b)

            @pl.when(g < NG - 1)
            def _():
                c_loads(g + 1)
            return carry
        lax.fori_loop(1, NG, cgroup, 0)
        c_adds_wait()
        plsc.subcore_barrier()

        for s0, ln in rchunks:
            pltpu.sync_copy(tab_sh.at[pl.ds(tb + s0, ln)], zb.at[pl.ds(0, ln)])

            def rbody(i, carry, ln=ln):
                st = jnp.minimum(i * L, ln - L)
                v = zb[pl.ds(st, L)]
                rb[pl.ds(st, L)] = 1.0 / jnp.maximum(v, 1.0)
                return carry
            lax.fori_loop(0, (ln + L - 1) // L, rbody, 0)
            pltpu.sync_copy(rb.at[pl.ds(0, ln)], tab_sh.at[pl.ds(tb + s0, ln)])
        plsc.subcore_barrier()

        dn = lax.GatherDimensionNumbers(
            offset_dims=(), collapsed_slice_dims=(0,), start_index_map=(0,))

        def m_group_loads(g, p):
            for b in range(NB):
                s = p * NB + b
                eb = sid * EC + (g * NB + b) * CC
                pltpu.async_copy(src_hbm.at[pl.ds(eb, CC)], srcb.at[s],
                                 sem_l[p])
                pltpu.async_copy(dst_hbm.at[pl.ds(eb, CC)], dstb.at[s],
                                 sem_l[p])
                pltpu.async_copy(typ_hbm.at[pl.ds(eb, CC)], typb.at[s],
                                 sem_l[p])

        def m_group_loads_wait(p):
            for b in range(NB):
                s = p * NB + b
                for ref in (srcb, dstb, typb):
                    pltpu.make_async_copy(src_hbm.at[pl.ds(0, CC)], ref.at[s],
                                          sem_l[p]).wait()

        def m_keys(b):
            for q in range(CC // L):
                sl = pl.ds(q * L, L)
                t16 = typb[b, sl]
                keyb[b, sl] = t16 * n + dstb[b, sl]
                gidxb[b, sl] = (srcb[b, sl] * r + t16) * 2 + cid
                sidxb[b, sl] = dstb[b, sl]

        def m_gathers(b, p):
            pltpu.async_copy(tab_sh.at[keyb.at[b]], cbr.at[b], sem_c[p])
            pltpu.async_copy(h_hbm.at[gidxb.at[b]], msgb.at[b], sem_g[p])

        def m_group_gathers_wait(p):
            for b in range(NB):
                s = p * NB + b
                pltpu.make_async_copy(tab_sh.at[keyb.at[s]], cbr.at[s],
                                      sem_c[p]).wait()
                pltpu.make_async_copy(h_hbm.at[gidxb.at[s]], msgb.at[s],
                                      sem_g[p]).wait()

        def m_scale(b):
            def sbody(q, carry2):
                c16 = cbr[b, pl.ds(q * L, L)]
                for jj in range(L):
                    bc = lax.gather(
                        c16, jnp.full((L, 1), jj, jnp.int32), dn,
                        slice_sizes=(1,),
                        mode=lax.GatherScatterMode.PROMISE_IN_BOUNDS)
                    for v in range(DH // L):
                        sl = pl.ds(v * L, L)
                        msgb[b, q * L + jj, sl] = msgb[b, q * L + jj, sl] * bc
                return carry2
            lax.fori_loop(0, CC // L, sbody, 0)

        def m_scatter(b, p):
            pltpu.async_copy(msgb.at[b], out_sh.at[sidxb.at[b]], sem_s[p],
                             add=True)

        def m_group_scatters_wait(p):
            for b in range(NB):
                s = p * NB + b
                pltpu.make_async_copy(msgb.at[s], out_sh.at[sidxb.at[s]],
                                      sem_s[p]).wait()

        def m_stage_in(g, p, first=False):
            m_group_loads_wait(p)
            if not first:
                m_group_scatters_wait(p)
            for b in range(NB):
                s = p * NB + b
                m_keys(s)
                m_gathers(s, p)

        def m_stage_out(p):
            m_group_gathers_wait(p)
            for b in range(NB):
                s = p * NB + b
                m_scale(s)
                m_scatter(s, p)

        m_group_loads(0, 0)
        m_stage_in(0, 0, first=True)
        m_group_loads(1, 1)
        m_stage_in(1, 1, first=True)

        def mpair(gi, carry):
            g0 = 2 * gi
            m_group_loads(g0, 0)
            m_stage_out(0)
            m_stage_in(g0, 0)
            m_group_loads(g0 + 1, 1)
            m_stage_out(1)
            m_stage_in(g0 + 1, 1)
            return carry
        lax.fori_loop(1, NG // 2, mpair, 0)
        m_stage_out(0)
        m_stage_out(1)
        m_group_scatters_wait(0)
        m_group_scatters_wait(1)
        plsc.subcore_barrier()

        pltpu.sync_copy(out_sh.at[pl.ds(ob, OROWS)],
                        out_hbm.at[cid, pl.ds(ob, OROWS)])
        if OTAIL:
            @pl.when(sid == NS - 1)
            def _():
                pltpu.sync_copy(out_sh.at[pl.ds(NS * OROWS, OTAIL)],
                                out_hbm.at[cid, pl.ds(NS * OROWS, OTAIL)])

    return body(hflat, src, dst, typ)


def kernel(x, edge_index, edge_type, basis, att, self_weight, bias):
    n, din = x.shape
    dout = self_weight.shape[1]
    r = att.shape[0]
    b = basis.shape[0]
    e = edge_type.shape[0]
    dh = dout // 2
    f32 = jnp.float32

    w2 = pl.pallas_call(
        _w_body,
        out_shape=jax.ShapeDtypeStruct((r, din * dout), f32),
    )(att, basis.reshape(b, din * dout))
    wcat = jnp.transpose(w2.reshape(r, din, dout), (1, 0, 2)).reshape(
        din, r * dout)

    TN = 1000
    h = pl.pallas_call(
        _h_body,
        grid=(n // TN,),
        in_specs=[pl.BlockSpec((TN, din), lambda j: (j, 0)),
                  pl.BlockSpec((din, r * dout), lambda j: (0, 0))],
        out_specs=pl.BlockSpec((TN, r * dout), lambda j: (j, 0)),
        out_shape=jax.ShapeDtypeStruct((n, r * dout), f32),
    )(x, wcat)

    partials = _sc_rgcn(n, r, dout, e,
                        h.reshape(2 * r * n, dh),
                        edge_index[0], edge_index[1], edge_type)

    out = pl.pallas_call(
        _combine_body,
        grid=(n // TN,),
        in_specs=[pl.BlockSpec((TN, din), lambda j: (j, 0)),
                  pl.BlockSpec((din, dout), lambda j: (0, 0)),
                  pl.BlockSpec((2, TN, dh), lambda j: (0, j, 0)),
                  pl.BlockSpec((1, dout), lambda j: (0, 0))],
        out_specs=pl.BlockSpec((TN, dout), lambda j: (j, 0)),
        out_shape=jax.ShapeDtypeStruct((n, dout), f32),
    )(x, self_weight, partials, bias.reshape(1, dout))
    return out


# speedup vs baseline: 88.4712x; 1.1662x over previous
import functools

import jax
import jax.numpy as jnp
from jax import lax
from jax.experimental import pallas as pl
from jax.experimental.pallas import tpu as pltpu
from jax.experimental.pallas import tpu_sc as plsc

_PREC = lax.Precision.DEFAULT


def _w_body(att_ref, basis_ref, w_ref):
    w_ref[...] = lax.dot_general(
        att_ref[...], basis_ref[...], (((1,), (0,)), ((), ())),
        preferred_element_type=jnp.float32, precision=_PREC)


def _h_body(x_ref, w_ref, h_ref):
    h_ref[...] = jnp.dot(x_ref[...], w_ref[...],
                         preferred_element_type=jnp.float32,
                         precision=_PREC)


def _combine_body(x_ref, sw_ref, p_ref, b_ref, o_ref):
    o_ref[...] = (jnp.concatenate([p_ref[0], p_ref[1]], axis=1) + b_ref[...]
                  + jnp.dot(x_ref[...], sw_ref[...],
                            preferred_element_type=jnp.float32, precision=_PREC))


def _sc_rgcn(n, r, d, e, hflat, src, dst, typ):
    NC, NS, L = 2, 16, 16
    RN = r * n
    DH = d // NC
    CC = 80
    NB = 5
    EC = e // NS
    NG = EC // CC // NB
    TAB = RN // NS
    OROWS = (n // NS) // 8 * 8
    OTAIL = n - OROWS * NS
    ZB = 1024
    ztab = list(range(0, TAB - ZB, ZB)) + [TAB - ZB]
    zout = list(range(0, OROWS - CC, CC)) + [OROWS - CC]
    rchunks = [(s, min(ZB, TAB - s)) for s in range(0, TAB, ZB)]
    mesh = plsc.VectorSubcoreMesh(core_axis_name="c", subcore_axis_name="s")

    @functools.partial(
        pl.kernel,
        out_type=jax.ShapeDtypeStruct((NC, n, DH), jnp.float32),
        mesh=mesh,
        compiler_params=pltpu.CompilerParams(use_tc_tiling_on_sc=False),
        scratch_types=[
            pltpu.VMEM_SHARED((RN,), jnp.float32),
            pltpu.VMEM_SHARED((n, DH), jnp.float32),
            pltpu.VMEM((2 * NB, CC), jnp.int32),
            pltpu.VMEM((2 * NB, CC), jnp.int32),
            pltpu.VMEM((2 * NB, CC), jnp.int32),
            pltpu.VMEM((2 * NB, CC), jnp.int32),
            pltpu.VMEM((2 * NB, CC), jnp.int32),
            pltpu.VMEM((2 * NB, CC), jnp.int32),
            pltpu.VMEM((2 * NB, CC), jnp.float32),
            pltpu.VMEM((2 * NB, CC, DH), jnp.float32),
            pltpu.VMEM((CC,), jnp.float32),
            pltpu.VMEM((ZB,), jnp.float32),
            pltpu.VMEM((ZB,), jnp.float32),
        ] + [pltpu.SemaphoreType.DMA] * 8,
    )
    def body(h_hbm, src_hbm, dst_hbm, typ_hbm, out_hbm,
             tab_sh, out_sh, srcb, dstb, typb, keyb, gidxb, sidxb, cbr, msgb,
             onesb, zb, rb, *sems):
        sem_l = sems[0:2]
        sem_g = sems[2:4]
        sem_c = sems[4:6]
        sem_s = sems[6:8]
        cid = lax.axis_index("c")
        sid = lax.axis_index("s")
        zeros = jnp.zeros((L,), jnp.float32)
        for i in range(ZB // L):
            zb[pl.ds(i * L, L)] = zeros

        def zrow(j, carry):
            for v in range(DH // L):
                msgb[0, j, pl.ds(v * L, L)] = zeros
            return carry
        lax.fori_loop(0, CC, zrow, 0)

        tb = sid * TAB
        for s0 in ztab:
            pltpu.sync_copy(zb, tab_sh.at[pl.ds(tb + s0, ZB)])
        ob = sid * OROWS
        for s0 in zout:
            pltpu.sync_copy(msgb.at[0], out_sh.at[pl.ds(ob + s0, CC)])
        if OTAIL:
            @pl.when(sid == NS - 1)
            def _():
                pltpu.sync_copy(msgb.at[0, pl.ds(0, OTAIL)],
                                out_sh.at[pl.ds(NS * OROWS, OTAIL)])
        plsc.subcore_barrier()

        ones = jnp.ones((L,), jnp.float32)
        for i in range(CC // L):
            onesb[pl.ds(i * L, L)] = ones

        def c_loads(g):
            for b in range(NB):
                eb = sid * EC + (g * NB + b) * CC
                pltpu.async_copy(typ_hbm.at[pl.ds(eb, CC)], typb.at[b],
                                 sem_l[0])
                pltpu.async_copy(dst_hbm.at[pl.ds(eb, CC)], dstb.at[b],
                                 sem_l[0])

        def c_loads_wait():
            for b in range(NB):
                pltpu.make_async_copy(typ_hbm.at[pl.ds(0, CC)], typb.at[b],
                                      sem_l[0]).wait()
                pltpu.make_async_copy(dst_hbm.at[pl.ds(0, CC)], dstb.at[b],
                                      sem_l[0]).wait()

        def c_keys(b):
            for q in range(CC // L):
                sl = pl.ds(q * L, L)
                keyb[b, sl] = typb[b, sl] * n + dstb[b, sl]

        def c_add(b):
            pltpu.async_copy(onesb, tab_sh.at[keyb.at[b]], sem_s[0], add=True)

        def c_adds_wait():
            for b in range(NB):
                pltpu.make_async_copy(onesb, tab_sh.at[keyb.at[b]],
                                      sem_s[0]).wait()

        c_loads(0)
        c_loads_wait()
        for b in range(NB):
            c_keys(b)
            c_add(b)
        c_loads(1)

        def cgroup(g, carry):
            c_loads_wait()
            c_adds_wait()
            for b in range(NB):
                c_keys(b)
                c_add(b)

            @pl.when(g < NG - 1)
            def _():
                c_loads(g + 1)
            return carry
        lax.fori_loop(1, NG, cgroup, 0)
        c_adds_wait()
        plsc.subcore_barrier()

        for s0, ln in rchunks:
            pltpu.sync_copy(tab_sh.at[pl.ds(tb + s0, ln)], zb.at[pl.ds(0, ln)])

            def rbody(i, carry, ln=ln):
                st = jnp.minimum(i * L, ln - L)
                v = zb[pl.ds(st, L)]
                rb[pl.ds(st, L)] = 1.0 / jnp.maximum(v, 1.0)
                return carry
            lax.fori_loop(0, (ln + L - 1) // L, rbody, 0)
            pltpu.sync_copy(rb.at[pl.ds(0, ln)], tab_sh.at[pl.ds(tb + s0, ln)])
        plsc.subcore_barrier()

        dn = lax.GatherDimensionNumbers(
            offset_dims=(), collapsed_slice_dims=(0,), start_index_map=(0,))

        def m_group_loads(g, p):
            for b in range(NB):
                s = p * NB + b
                eb = sid * EC + (g * NB + b) * CC
                pltpu.async_copy(src_hbm.at[pl.ds(eb, CC)], srcb.at[s],
                                 sem_l[p])
                pltpu.async_copy(dst_hbm.at[pl.ds(eb, CC)], dstb.at[s],
                                 sem_l[p])
                pltpu.async_copy(typ_hbm.at[pl.ds(eb, CC)], typb.at[s],
                                 sem_l[p])

        def m_group_loads_wait(p):
            for b in range(NB):
                s = p * NB + b
                for ref in (srcb, dstb, typb):
                    pltpu.make_async_copy(src_hbm.at[pl.ds(0, CC)], ref.at[s],
                                          sem_l[p]).wait()

        def m_keys(b):
            for q in range(CC // L):
                sl = pl.ds(q * L, L)
                t16 = typb[b, sl]
                keyb[b, sl] = t16 * n + dstb[b, sl]
                gidxb[b, sl] = (srcb[b, sl] * r + t16) * 2 + cid
                sidxb[b, sl] = dstb[b, sl]

        def m_gathers(b, p):
            pltpu.async_copy(tab_sh.at[keyb.at[b]], cbr.at[b], sem_c[p])
            pltpu.async_copy(h_hbm.at[gidxb.at[b]], msgb.at[b], sem_g[p])

        def m_group_gathers_wait(p):
            for b in range(NB):
                s = p * NB + b
                pltpu.make_async_copy(tab_sh.at[keyb.at[s]], cbr.at[s],
                                      sem_c[p]).wait()
                pltpu.make_async_copy(h_hbm.at[gidxb.at[s]], msgb.at[s],
                                      sem_g[p]).wait()

        def m_scale(b):
            @plsc.parallel_loop(0, CC // L, 1)
            def sbody(q):
                c16 = cbr[b, pl.ds(q * L, L)]
                for jj in range(L):
                    bc = lax.gather(
                        c16, jnp.full((L, 1), jj, jnp.int32), dn,
                        slice_sizes=(1,),
                        mode=lax.GatherScatterMode.PROMISE_IN_BOUNDS)
                    for v in range(DH // L):
                        sl = pl.ds(v * L, L)
                        msgb[b, q * L + jj, sl] = msgb[b, q * L + jj, sl] * bc

        def m_scatter(b, p):
            pltpu.async_copy(msgb.at[b], out_sh.at[sidxb.at[b]], sem_s[p],
                             add=True)

        def m_group_scatters_wait(p):
            for b in range(NB):
                s = p * NB + b
                pltpu.make_async_copy(msgb.at[s], out_sh.at[sidxb.at[s]],
                                      sem_s[p]).wait()

        def m_stage_in(g, p, first=False):
            m_group_loads_wait(p)
            if not first:
                m_group_scatters_wait(p)
            for b in range(NB):
                s = p * NB + b
                m_keys(s)
                m_gathers(s, p)

        def m_stage_out(p):
            m_group_gathers_wait(p)
            for b in range(NB):
                s = p * NB + b
                m_scale(s)
                m_scatter(s, p)

        m_group_loads(0, 0)
        m_stage_in(0, 0, first=True)
        m_group_loads(1, 1)
        m_stage_in(1, 1, first=True)

        def mpair(gi, carry):
            g0 = 2 * gi
            m_group_loads(g0, 0)
            m_stage_out(0)
            m_stage_in(g0, 0)
            m_group_loads(g0 + 1, 1)
            m_stage_out(1)
            m_stage_in(g0 + 1, 1)
            return carry
        lax.fori_loop(1, NG // 2, mpair, 0)
        m_stage_out(0)
        m_stage_out(1)
        m_group_scatters_wait(0)
        m_group_scatters_wait(1)
        plsc.subcore_barrier()

        pltpu.sync_copy(out_sh.at[pl.ds(ob, OROWS)],
                        out_hbm.at[cid, pl.ds(ob, OROWS)])
        if OTAIL:
            @pl.when(sid == NS - 1)
            def _():
                pltpu.sync_copy(out_sh.at[pl.ds(NS * OROWS, OTAIL)],
                                out_hbm.at[cid, pl.ds(NS * OROWS, OTAIL)])

    return body(hflat, src, dst, typ)


def kernel(x, edge_index, edge_type, basis, att, self_weight, bias):
    n, din = x.shape
    dout = self_weight.shape[1]
    r = att.shape[0]
    b = basis.shape[0]
    e = edge_type.shape[0]
    dh = dout // 2
    f32 = jnp.float32

    w2 = pl.pallas_call(
        _w_body,
        out_shape=jax.ShapeDtypeStruct((r, din * dout), f32),
    )(att, basis.reshape(b, din * dout))
    wcat = jnp.transpose(w2.reshape(r, din, dout), (1, 0, 2)).reshape(
        din, r * dout)

    TN = 1000
    h = pl.pallas_call(
        _h_body,
        grid=(n // TN,),
        in_specs=[pl.BlockSpec((TN, din), lambda j: (j, 0)),
                  pl.BlockSpec((din, r * dout), lambda j: (0, 0))],
        out_specs=pl.BlockSpec((TN, r * dout), lambda j: (j, 0)),
        out_shape=jax.ShapeDtypeStruct((n, r * dout), f32),
    )(x, wcat)

    partials = _sc_rgcn(n, r, dout, e,
                        h.reshape(2 * r * n, dh),
                        edge_index[0], edge_index[1], edge_type)

    out = pl.pallas_call(
        _combine_body,
        grid=(n // TN,),
        in_specs=[pl.BlockSpec((TN, din), lambda j: (j, 0)),
                  pl.BlockSpec((din, dout), lambda j: (0, 0)),
                  pl.BlockSpec((2, TN, dh), lambda j: (0, j, 0)),
                  pl.BlockSpec((1, dout), lambda j: (0, 0))],
        out_specs=pl.BlockSpec((TN, dout), lambda j: (j, 0)),
        out_shape=jax.ShapeDtypeStruct((n, dout), f32),
    )(x, self_weight, partials, bias.reshape(1, dout))
    return out


# speedup vs baseline: 94.4444x; 1.0675x over previous
import functools

import jax
import jax.numpy as jnp
from jax import lax
from jax.experimental import pallas as pl
from jax.experimental.pallas import tpu as pltpu
from jax.experimental.pallas import tpu_sc as plsc

_PREC = lax.Precision.DEFAULT


def _w_body(att_ref, basis_ref, w_ref):
    w_ref[...] = lax.dot_general(
        att_ref[...], basis_ref[...], (((1,), (0,)), ((), ())),
        preferred_element_type=jnp.float32, precision=_PREC)


def _h_body(x_ref, w_ref, h_ref):
    h_ref[...] = jnp.dot(x_ref[...], w_ref[...],
                         preferred_element_type=jnp.float32,
                         precision=_PREC)


def _combine_body(x_ref, sw_ref, p_ref, b_ref, o_ref):
    o_ref[...] = (jnp.concatenate([p_ref[0], p_ref[1]], axis=1) + b_ref[...]
                  + jnp.dot(x_ref[...], sw_ref[...],
                            preferred_element_type=jnp.float32, precision=_PREC))


def _sc_rgcn(n, r, d, e, hflat, src, dst, typ):
    NC, NS, L = 2, 16, 16
    RN = r * n
    DH = d // NC
    CC = 80
    NB = 5
    EC = e // NS
    NG = EC // CC // NB
    TAB = RN // NS
    OROWS = (n // NS) // 8 * 8
    OTAIL = n - OROWS * NS
    ZB = 1024
    ztab = list(range(0, TAB - ZB, ZB)) + [TAB - ZB]
    zout = list(range(0, OROWS - CC, CC)) + [OROWS - CC]
    rchunks = [(s, min(ZB, TAB - s)) for s in range(0, TAB, ZB)]
    mesh = plsc.VectorSubcoreMesh(core_axis_name="c", subcore_axis_name="s")

    @functools.partial(
        pl.kernel,
        out_type=jax.ShapeDtypeStruct((NC, n, DH), jnp.float32),
        mesh=mesh,
        compiler_params=pltpu.CompilerParams(use_tc_tiling_on_sc=False),
        scratch_types=[
            pltpu.VMEM_SHARED((RN,), jnp.float32),
            pltpu.VMEM_SHARED((n, DH), jnp.float32),
            pltpu.VMEM((2 * NB, CC), jnp.int32),
            pltpu.VMEM((2 * NB, CC), jnp.int32),
            pltpu.VMEM((2 * NB, CC), jnp.int32),
            pltpu.VMEM((2 * NB, CC), jnp.int32),
            pltpu.VMEM((2 * NB, CC), jnp.int32),
            pltpu.VMEM((2 * NB, CC), jnp.int32),
            pltpu.VMEM((2 * NB, CC), jnp.float32),
            pltpu.VMEM((2 * NB, CC, DH), jnp.float32),
            pltpu.VMEM((CC,), jnp.float32),
            pltpu.VMEM((ZB,), jnp.float32),
            pltpu.VMEM((ZB,), jnp.float32),
        ] + [pltpu.SemaphoreType.DMA] * 8,
    )
    def body(h_hbm, src_hbm, dst_hbm, typ_hbm, out_hbm,
             tab_sh, out_sh, srcb, dstb, typb, keyb, gidxb, sidxb, cbr, msgb,
             onesb, zb, rb, *sems):
        sem_l = sems[0:2]
        sem_g = sems[2:4]
        sem_c = sems[4:6]
        sem_s = sems[6:8]
        cid = lax.axis_index("c")
        sid = lax.axis_index("s")
        zeros = jnp.zeros((L,), jnp.float32)
        for i in range(ZB // L):
            zb[pl.ds(i * L, L)] = zeros

        def zrow(j, carry):
            for v in range(DH // L):
                msgb[0, j, pl.ds(v * L, L)] = zeros
            return carry
        lax.fori_loop(0, CC, zrow, 0)

        tb = sid * TAB
        for s0 in ztab:
            pltpu.sync_copy(zb, tab_sh.at[pl.ds(tb + s0, ZB)])
        ob = sid * OROWS
        for s0 in zout:
            pltpu.sync_copy(msgb.at[0], out_sh.at[pl.ds(ob + s0, CC)])
        if OTAIL:
            @pl.when(sid == NS - 1)
            def _():
                pltpu.sync_copy(msgb.at[0, pl.ds(0, OTAIL)],
                                out_sh.at[pl.ds(NS * OROWS, OTAIL)])
        plsc.subcore_barrier()

        ones = jnp.ones((L,), jnp.float32)
        for i in range(CC // L):
            onesb[pl.ds(i * L, L)] = ones

        def c_loads(g):
            for b in range(NB):
                eb = sid * EC + (g * NB + b) * CC
                pltpu.async_copy(typ_hbm.at[pl.ds(eb, CC)], typb.at[b],
                                 sem_l[0])
                pltpu.async_copy(dst_hbm.at[pl.ds(eb, CC)], dstb.at[b],
                                 sem_l[0])

        def c_loads_wait():
            for b in range(NB):
                pltpu.make_async_copy(typ_hbm.at[pl.ds(0, CC)], typb.at[b],
                                      sem_l[0]).wait()
                pltpu.make_async_copy(dst_hbm.at[pl.ds(0, CC)], dstb.at[b],
                                      sem_l[0]).wait()

        def c_keys(b):
            for q in range(CC // L):
                sl = pl.ds(q * L, L)
                keyb[b, sl] = typb[b, sl] * n + dstb[b, sl]

        def c_add(b):
            pltpu.async_copy(onesb, tab_sh.at[keyb.at[b]], sem_s[0], add=True)

        def c_adds_wait():
            for b in range(NB):
                pltpu.make_async_copy(onesb, tab_sh.at[keyb.at[b]],
                                      sem_s[0]).wait()

        c_loads(0)
        c_loads_wait()
        for b in range(NB):
            c_keys(b)
            c_add(b)
        c_loads(1)

        def cgroup(g, carry):
            c_loads_wait()
            c_adds_wait()
            for b in range(NB):
                c_keys(b)
                c_add(b)

            @pl.when(g < NG - 1)
            def _():
                c_loads(g + 1)
            return carry
        lax.fori_loop(1, NG, cgroup, 0)
        c_adds_wait()
        plsc.subcore_barrier()

        for s0, ln in rchunks:
            pltpu.sync_copy(tab_sh.at[pl.ds(tb + s0, ln)], zb.at[pl.ds(0, ln)])

            def rbody(i, carry, ln=ln):
                st = jnp.minimum(i * L, ln - L)
                v = zb[pl.ds(st, L)]
                rb[pl.ds(st, L)] = 1.0 / jnp.maximum(v, 1.0)
                return carry
            lax.fori_loop(0, (ln + L - 1) // L, rbody, 0)
            pltpu.sync_copy(rb.at[pl.ds(0, ln)], tab_sh.at[pl.ds(tb + s0, ln)])
        plsc.subcore_barrier()

        dn = lax.GatherDimensionNumbers(
            offset_dims=(), collapsed_slice_dims=(0,), start_index_map=(0,))

        def m_group_loads(g, p):
            for b in range(NB):
                s = p * NB + b
                eb = sid * EC + (g * NB + b) * CC
                pltpu.async_copy(src_hbm.at[pl.ds(eb, CC)], srcb.at[s],
                                 sem_l[p])
                pltpu.async_copy(dst_hbm.at[pl.ds(eb, CC)], dstb.at[s],
                                 sem_l[p])
                pltpu.async_copy(typ_hbm.at[pl.ds(eb, CC)], typb.at[s],
                                 sem_l[p])

        def m_group_loads_wait(p):
            for b in range(NB):
                s = p * NB + b
                for ref in (srcb, dstb, typb):
                    pltpu.make_async_copy(src_hbm.at[pl.ds(0, CC)], ref.at[s],
                                          sem_l[p]).wait()

        def m_keys(b):
            for q in range(CC // L):
                sl = pl.ds(q * L, L)
                t16 = typb[b, sl]
                keyb[b, sl] = t16 * n + dstb[b, sl]
                gidxb[b, sl] = (srcb[b, sl] * r + t16) * 2 + cid
                sidxb[b, sl] = dstb[b, sl]

        def m_gathers(b, p):
            pltpu.async_copy(tab_sh.at[keyb.at[b]], cbr.at[b], sem_c[p])
            pltpu.async_copy(h_hbm.at[gidxb.at[b]], msgb.at[b], sem_g[p])

        def m_group_gathers_wait(p):
            for b in range(NB):
                s = p * NB + b
                pltpu.make_async_copy(tab_sh.at[keyb.at[s]], cbr.at[s],
                                      sem_c[p]).wait()
                pltpu.make_async_copy(h_hbm.at[gidxb.at[s]], msgb.at[s],
                                      sem_g[p]).wait()

        def m_scale(b):
            @plsc.parallel_loop(0, CC // L, 1, unroll=2)
            def sbody(q):
                c16 = cbr[b, pl.ds(q * L, L)]
                for jj in range(L):
                    bc = lax.gather(
                        c16, jnp.full((L, 1), jj, jnp.int32), dn,
                        slice_sizes=(1,),
                        mode=lax.GatherScatterMode.PROMISE_IN_BOUNDS)
                    for v in range(DH // L):
                        sl = pl.ds(v * L, L)
                        msgb[b, q * L + jj, sl] = msgb[b, q * L + jj, sl] * bc

        def m_scatter(b, p):
            pltpu.async_copy(msgb.at[b], out_sh.at[sidxb.at[b]], sem_s[p],
                             add=True)

        def m_group_scatters_wait(p):
            for b in range(NB):
                s = p * NB + b
                pltpu.make_async_copy(msgb.at[s], out_sh.at[sidxb.at[s]],
                                      sem_s[p]).wait()

        def m_stage_in(g, p, first=False):
            m_group_loads_wait(p)
            if not first:
                m_group_scatters_wait(p)
            for b in range(NB):
                s = p * NB + b
                m_keys(s)
                m_gathers(s, p)

        def m_stage_out(p):
            m_group_gathers_wait(p)
            for b in range(NB):
                s = p * NB + b
                m_scale(s)
                m_scatter(s, p)

        m_group_loads(0, 0)
        m_stage_in(0, 0, first=True)
        m_group_loads(1, 1)
        m_stage_in(1, 1, first=True)

        def mpair(gi, carry):
            g0 = 2 * gi
            m_group_loads(g0, 0)
            m_stage_out(0)
            m_stage_in(g0, 0)
            m_group_loads(g0 + 1, 1)
            m_stage_out(1)
            m_stage_in(g0 + 1, 1)
            return carry
        lax.fori_loop(1, NG // 2, mpair, 0)
        m_stage_out(0)
        m_stage_out(1)
        m_group_scatters_wait(0)
        m_group_scatters_wait(1)
        plsc.subcore_barrier()

        pltpu.sync_copy(out_sh.at[pl.ds(ob, OROWS)],
                        out_hbm.at[cid, pl.ds(ob, OROWS)])
        if OTAIL:
            @pl.when(sid == NS - 1)
            def _():
                pltpu.sync_copy(out_sh.at[pl.ds(NS * OROWS, OTAIL)],
                                out_hbm.at[cid, pl.ds(NS * OROWS, OTAIL)])

    return body(hflat, src, dst, typ)


def kernel(x, edge_index, edge_type, basis, att, self_weight, bias):
    n, din = x.shape
    dout = self_weight.shape[1]
    r = att.shape[0]
    b = basis.shape[0]
    e = edge_type.shape[0]
    dh = dout // 2
    f32 = jnp.float32

    w2 = pl.pallas_call(
        _w_body,
        out_shape=jax.ShapeDtypeStruct((r, din * dout), f32),
    )(att, basis.reshape(b, din * dout))
    wcat = jnp.transpose(w2.reshape(r, din, dout), (1, 0, 2)).reshape(
        din, r * dout)

    TN = 1000
    h = pl.pallas_call(
        _h_body,
        grid=(n // TN,),
        in_specs=[pl.BlockSpec((TN, din), lambda j: (j, 0)),
                  pl.BlockSpec((din, r * dout), lambda j: (0, 0))],
        out_specs=pl.BlockSpec((TN, r * dout), lambda j: (j, 0)),
        out_shape=jax.ShapeDtypeStruct((n, r * dout), f32),
    )(x, wcat)

    partials = _sc_rgcn(n, r, dout, e,
                        h.reshape(2 * r * n, dh),
                        edge_index[0], edge_index[1], edge_type)

    out = pl.pallas_call(
        _combine_body,
        grid=(n // TN,),
        in_specs=[pl.BlockSpec((TN, din), lambda j: (j, 0)),
                  pl.BlockSpec((din, dout), lambda j: (0, 0)),
                  pl.BlockSpec((2, TN, dh), lambda j: (0, j, 0)),
                  pl.BlockSpec((1, dout), lambda j: (0, 0))],
        out_specs=pl.BlockSpec((TN, dout), lambda j: (j, 0)),
        out_shape=jax.ShapeDtypeStruct((n, dout), f32),
    )(x, self_weight, partials, bias.reshape(1, dout))
    return out
